# baseline (speedup 1.0000x reference)
import functools

import jax
import jax.numpy as jnp
from jax import lax
from jax.experimental import pallas as pl
from jax.experimental.pallas import tpu as pltpu
from jax.experimental.pallas import tpu_sc as plsc

_NUM_CORES = 2
_NUM_SUBCORES = 16
_NUM_WORKERS = _NUM_CORES * _NUM_SUBCORES
_CHUNK = 256
_R = 2048
_N_TC = 147456


def _sc_gather(table, idx_flat, n, d):
    mesh = plsc.VectorSubcoreMesh(core_axis_name="c", subcore_axis_name="s")
    b_per_w = n // _NUM_WORKERS
    chunks = b_per_w // _CHUNK
    assert chunks % 2 == 0 and chunks >= 4

    @functools.partial(
        pl.kernel,
        out_type=jax.ShapeDtypeStruct((n, d), table.dtype),
        mesh=mesh,
        scratch_types=[
            pltpu.VMEM((b_per_w,), jnp.int32),
            pltpu.VMEM((_CHUNK, d), table.dtype),
            pltpu.VMEM((_CHUNK, d), table.dtype),
            pltpu.SemaphoreType.DMA,
            pltpu.SemaphoreType.DMA,
            pltpu.SemaphoreType.DMA,
            pltpu.SemaphoreType.DMA,
        ],
    )
    def gather_kernel(table_hbm, idx_hbm, out_hbm, idx_v, rows0, rows1,
                      gsem0, gsem1, ssem0, ssem1):
        rows = (rows0, rows1)
        gsem = (gsem0, gsem1)
        ssem = (ssem0, ssem1)
        wid = lax.axis_index("s") * _NUM_CORES + lax.axis_index("c")
        base = wid * b_per_w
        pltpu.sync_copy(idx_hbm.at[pl.ds(base, b_per_w)], idx_v)

        def start_gather(b, c):
            pltpu.async_copy(
                table_hbm.at[idx_v.at[pl.ds(c * _CHUNK, _CHUNK)]],
                rows[b], gsem[b])

        def wait_gather(b):
            pltpu.make_async_copy(
                table_hbm.at[idx_v.at[pl.ds(0, _CHUNK)]],
                rows[b], gsem[b]).wait()

        def start_store(b, c):
            pltpu.async_copy(
                rows[b], out_hbm.at[pl.ds(base + c * _CHUNK, _CHUNK)],
                ssem[b])

        def wait_store(b):
            pltpu.make_async_copy(
                rows[b], out_hbm.at[pl.ds(base, _CHUNK)], ssem[b]).wait()

        start_gather(0, 0)
        start_gather(1, 1)

        @pl.loop(0, chunks - 2, step=2)
        def _(c0):
            for b in range(2):
                wait_gather(b)
                start_store(b, c0 + b)
            for b in range(2):
                wait_store(b)
                start_gather(b, c0 + 2 + b)

        for b in range(2):
            wait_gather(b)
            start_store(b, chunks - 2 + b)
        for b in range(2):
            wait_store(b)

    return gather_kernel(table, idx_flat)


def _tc_gather(table, idx2d, n, d):
    v = table.shape[0]
    grid = (n // _R,)

    def body(idx_ref, table_hbm, out_ref, table_v, idx_s, sem_t, sem_i):
        i = pl.program_id(0)

        @pl.when(i == 0)
        def _():
            pltpu.async_copy(table_hbm, table_v, sem_t).wait()

        pltpu.async_copy(idx_ref, idx_s, sem_i).wait()

        def step(r, _):
            k = idx_s[0, r]
            out_ref[pl.ds(r, 1), :] = table_v[pl.ds(k, 1), :]
            return 0

        lax.fori_loop(0, _R, step, 0, unroll=16)

    return pl.pallas_call(
        body,
        grid=grid,
        in_specs=[
            pl.BlockSpec((1, _R), lambda i: (0, i)),
            pl.BlockSpec(memory_space=pl.ANY),
        ],
        out_specs=pl.BlockSpec((_R, d), lambda i: (i, 0)),
        out_shape=jax.ShapeDtypeStruct((n, d), jnp.float32),
        scratch_shapes=[
            pltpu.VMEM((v, d), jnp.float32),
            pltpu.SMEM((1, _R), jnp.int32),
            pltpu.SemaphoreType.DMA,
            pltpu.SemaphoreType.DMA,
        ],
    )(idx2d, table)


def kernel(x, table):
    b, l = x.shape
    v, d = table.shape
    n = b * l
    n_sc = n - _N_TC
    idx_flat = x.reshape(n)
    sc_out = _sc_gather(table, idx_flat[:n_sc], n_sc, d)
    tc_out = _tc_gather(table, idx_flat[n_sc:].reshape(1, _N_TC), _N_TC, d)
    out = jnp.concatenate([sc_out, tc_out], axis=0)
    return out.reshape(b, l, d)


# speedup vs baseline: 1.0033x; 1.0033x over previous
import functools

import jax
import jax.numpy as jnp
from jax import lax
from jax.experimental import pallas as pl
from jax.experimental.pallas import tpu as pltpu
from jax.experimental.pallas import tpu_sc as plsc

_NUM_CORES = 2
_NUM_SUBCORES = 16
_NUM_WORKERS = _NUM_CORES * _NUM_SUBCORES
_CHUNK = 256
_R = 2048
_N_TC = 147456


def _sc_gather(table, idx_flat, n, d):
    mesh = plsc.VectorSubcoreMesh(core_axis_name="c", subcore_axis_name="s")
    b_per_w = n // _NUM_WORKERS
    chunks = b_per_w // _CHUNK
    assert chunks % 2 == 0 and chunks >= 4

    @functools.partial(
        pl.kernel,
        out_type=jax.ShapeDtypeStruct((n, d), table.dtype),
        mesh=mesh,
        scratch_types=[
            pltpu.VMEM((b_per_w,), jnp.int32),
            pltpu.VMEM((_CHUNK, d), table.dtype),
            pltpu.VMEM((_CHUNK, d), table.dtype),
            pltpu.SemaphoreType.DMA,
            pltpu.SemaphoreType.DMA,
            pltpu.SemaphoreType.DMA,
            pltpu.SemaphoreType.DMA,
        ],
    )
    def gather_kernel(table_hbm, idx_hbm, out_hbm, idx_v, rows0, rows1,
                      gsem0, gsem1, ssem0, ssem1):
        rows = (rows0, rows1)
        gsem = (gsem0, gsem1)
        ssem = (ssem0, ssem1)
        wid = lax.axis_index("s") * _NUM_CORES + lax.axis_index("c")
        base = wid * b_per_w
        pltpu.sync_copy(idx_hbm.at[pl.ds(base, b_per_w)], idx_v)

        def start_gather(b, c):
            pltpu.async_copy(
                table_hbm.at[idx_v.at[pl.ds(c * _CHUNK, _CHUNK)]],
                rows[b], gsem[b])

        def wait_gather(b):
            pltpu.make_async_copy(
                table_hbm.at[idx_v.at[pl.ds(0, _CHUNK)]],
                rows[b], gsem[b]).wait()

        def start_store(b, c):
            pltpu.async_copy(
                rows[b], out_hbm.at[pl.ds(base + c * _CHUNK, _CHUNK)],
                ssem[b])

        def wait_store(b):
            pltpu.make_async_copy(
                rows[b], out_hbm.at[pl.ds(base, _CHUNK)], ssem[b]).wait()

        start_gather(0, 0)
        start_gather(1, 1)

        @pl.loop(0, chunks - 2, step=2)
        def _(c0):
            for b in range(2):
                wait_gather(b)
                start_store(b, c0 + b)
            for b in range(2):
                wait_store(b)
                start_gather(b, c0 + 2 + b)

        for b in range(2):
            wait_gather(b)
            start_store(b, chunks - 2 + b)
        for b in range(2):
            wait_store(b)

    return gather_kernel(table, idx_flat)


def _tc_gather(table, idx2d, n, d):
    v = table.shape[0]
    grid = (n // _R,)

    def body(idx_ref, table_hbm, out_ref, table_v, idx_s, sem_t, sem_i):
        i = pl.program_id(0)

        @pl.when(i == 0)
        def _():
            pltpu.async_copy(table_hbm, table_v, sem_t).wait()

        pltpu.async_copy(idx_ref, idx_s, sem_i).wait()

        def step(r, _):
            k = idx_s[0, r]
            out_ref[pl.ds(r, 1), :] = table_v[pl.ds(k, 1), :]
            return 0

        lax.fori_loop(0, _R, step, 0, unroll=16)

    return pl.pallas_call(
        body,
        grid=grid,
        in_specs=[
            pl.BlockSpec((1, _R), lambda i: (0, i)),
            pl.BlockSpec(memory_space=pl.ANY),
        ],
        out_specs=pl.BlockSpec((_R, d), lambda i: (i, 0)),
        out_shape=jax.ShapeDtypeStruct((n, d), jnp.float32),
        scratch_shapes=[
            pltpu.VMEM((v, d), jnp.float32),
            pltpu.SMEM((1, _R), jnp.int32),
            pltpu.SemaphoreType.DMA,
            pltpu.SemaphoreType.DMA,
        ],
    )(idx2d, table)


def kernel(x, table):
    b, l = x.shape
    v, d = table.shape
    n = b * l
    n_sc = n - _N_TC
    idx_flat = x.reshape(n)
    tc_out = _tc_gather(table, idx_flat[n_sc:].reshape(1, _N_TC), _N_TC, d)
    sc_out = _sc_gather(table, idx_flat[:n_sc], n_sc, d)
    out = jnp.concatenate([sc_out, tc_out], axis=0)
    return out.reshape(b, l, d)


# speedup vs baseline: 1.7957x; 1.7897x over previous
import functools

import jax
import jax.numpy as jnp
from jax import lax
from jax.experimental import pallas as pl
from jax.experimental.pallas import tpu as pltpu
from jax.experimental.pallas import tpu_sc as plsc

_NUM_CORES = 2
_NUM_SUBCORES = 16
_NUM_WORKERS = _NUM_CORES * _NUM_SUBCORES
_CHUNK = 200
_RING = 4


def _gather_rows(table, idx_flat, n, d):
    mesh = plsc.VectorSubcoreMesh(core_axis_name="c", subcore_axis_name="s")
    b_per_w = n // _NUM_WORKERS
    chunks = b_per_w // _CHUNK
    assert (chunks - 4) % _RING == 0 and chunks >= 8

    @functools.partial(
        pl.kernel,
        out_type=jax.ShapeDtypeStruct((n, d), table.dtype),
        mesh=mesh,
        scratch_types=(
            [pltpu.VMEM((b_per_w,), jnp.int32)]
            + [pltpu.VMEM((_CHUNK, d), table.dtype) for _ in range(_RING)]
            + [pltpu.SemaphoreType.DMA for _ in range(2 * _RING)]
        ),
    )
    def gather_kernel(table_hbm, idx_hbm, out_hbm, idx_v, *bufs):
        rows = bufs[:_RING]
        gsem = bufs[_RING:2 * _RING]
        ssem = bufs[2 * _RING:]
        wid = lax.axis_index("s") * _NUM_CORES + lax.axis_index("c")
        base = wid * b_per_w
        pltpu.sync_copy(idx_hbm.at[pl.ds(base, b_per_w)], idx_v)

        def start_gather(b, c):
            pltpu.async_copy(
                table_hbm.at[idx_v.at[pl.ds(c * _CHUNK, _CHUNK)]],
                rows[b], gsem[b])

        def wait_gather(b):
            pltpu.make_async_copy(
                table_hbm.at[idx_v.at[pl.ds(0, _CHUNK)]],
                rows[b], gsem[b]).wait()

        def start_store(b, c):
            pltpu.async_copy(
                rows[b], out_hbm.at[pl.ds(base + c * _CHUNK, _CHUNK)],
                ssem[b])

        def wait_store(b):
            pltpu.make_async_copy(
                rows[b], out_hbm.at[pl.ds(base, _CHUNK)], ssem[b]).wait()

        start_gather(0, 0)
        start_gather(1, 1)

        for c in (0, 1):
            wait_gather(c % _RING)
            start_store(c % _RING, c)
            start_gather((c + 2) % _RING, c + 2)

        @pl.loop(2, chunks - 2, step=_RING)
        def _(c0):
            for j in range(_RING):
                b = (2 + j) % _RING
                b2 = j % _RING
                c = c0 + j
                wait_gather(b)
                wait_store(b2)
                start_store(b, c)
                start_gather(b2, c + 2)

        for c in (chunks - 2, chunks - 1):
            wait_gather(c % _RING)
            wait_store((c + 2) % _RING)
            start_store(c % _RING, c)
        for c in (chunks - 2, chunks - 1):
            wait_store(c % _RING)

    return gather_kernel(table, idx_flat)


def kernel(x, table):
    b, l = x.shape
    v, d = table.shape
    n = b * l
    idx_flat = x.reshape(n)
    out = _gather_rows(table, idx_flat, n, d)
    return out.reshape(b, l, d)
